# baseline (device time: 53679 ns/iter reference)
import jax
import jax.numpy as jnp
from jax import lax
from jax.experimental import pallas as pl
from jax.experimental.pallas import tpu as pltpu


def kernel(x, Win0, Wout0, Win1, Wout1, Win2, Wout2):
    b, d_half = x.shape
    h_half = Win0.shape[1]
    bf16 = jnp.bfloat16

    x = x.astype(bf16)
    Win0, Win1, Win2 = (w.astype(bf16) for w in (Win0, Win1, Win2))
    Wout0, Wout1, Wout2 = (w.astype(bf16) for w in (Wout0, Wout1, Wout2))

    def body(x_ref, win0_ref, wout0_ref, win1_ref, wout1_ref, win2_ref,
             wout2_ref, out_ref,
             h_send, h_recv, x_send, x_recv,
             h_send_sems, h_recv_sems, x_send_sems, x_recv_sems):
        my_x = lax.axis_index("x")
        my_y = lax.axis_index("y")
        y_peer = (my_x, 1 - my_y)
        x_peer = (1 - my_x, my_y)

        barrier = pltpu.get_barrier_semaphore()
        for peer in (y_peer, x_peer):
            pl.semaphore_signal(
                barrier, inc=1, device_id=peer,
                device_id_type=pl.DeviceIdType.MESH,
            )
        pl.semaphore_wait(barrier, 2)

        def exchange(slot, partial_f32, send_buf, recv_buf, send_sems,
                     recv_sems, peer):
            send_buf[slot] = partial_f32.astype(bf16)
            rdma = pltpu.make_async_remote_copy(
                src_ref=send_buf.at[slot],
                dst_ref=recv_buf.at[slot],
                send_sem=send_sems.at[slot],
                recv_sem=recv_sems.at[slot],
                device_id=peer,
                device_id_type=pl.DeviceIdType.MESH,
            )
            rdma.start()
            rdma.wait()
            return partial_f32 + recv_buf[slot].astype(jnp.float32)

        acts = x_ref[...]
        layers = ((win0_ref, wout0_ref), (win1_ref, wout1_ref),
                  (win2_ref, wout2_ref))
        for layer, (win_ref, wout_ref) in enumerate(layers):
            hp = jnp.dot(acts, win_ref[...], preferred_element_type=jnp.float32)
            h = exchange(layer, hp, h_send, h_recv, h_send_sems,
                         h_recv_sems, y_peer)
            h = jnp.maximum(h, 0.0).astype(bf16)
            xp = jnp.dot(h, wout_ref[...], preferred_element_type=jnp.float32)
            xs = exchange(layer, xp, x_send, x_recv, x_send_sems,
                          x_recv_sems, x_peer)
            acts = xs.astype(bf16)
        out_ref[...] = xs

    return pl.pallas_call(
        body,
        out_shape=jax.ShapeDtypeStruct((b, d_half), jnp.float32),
        in_specs=[pl.BlockSpec(memory_space=pltpu.VMEM)] * 7,
        out_specs=pl.BlockSpec(memory_space=pltpu.VMEM),
        scratch_shapes=[
            pltpu.VMEM((3, b, h_half), bf16),
            pltpu.VMEM((3, b, h_half), bf16),
            pltpu.VMEM((3, b, d_half), bf16),
            pltpu.VMEM((3, b, d_half), bf16),
            pltpu.SemaphoreType.DMA((3,)),
            pltpu.SemaphoreType.DMA((3,)),
            pltpu.SemaphoreType.DMA((3,)),
            pltpu.SemaphoreType.DMA((3,)),
        ],
        compiler_params=pltpu.CompilerParams(collective_id=0),
    )(x, Win0, Wout0, Win1, Wout1, Win2, Wout2)


# device time: 50185 ns/iter; 1.0696x vs baseline; 1.0696x over previous
import jax
import jax.numpy as jnp
from jax import lax
from jax.experimental import pallas as pl
from jax.experimental.pallas import tpu as pltpu


def kernel(x, Win0, Wout0, Win1, Wout1, Win2, Wout2):
    b, d_half = x.shape
    h_half = Win0.shape[1]
    bf16 = jnp.bfloat16

    def body(x_ref, win0_ref, wout0_ref, win1_ref, wout1_ref, win2_ref,
             wout2_ref, out_ref,
             h_send, h_recv, x_send, x_recv,
             h_send_sems, h_recv_sems, x_send_sems, x_recv_sems):
        my_x = lax.axis_index("x")
        my_y = lax.axis_index("y")
        y_peer = (my_x, 1 - my_y)
        x_peer = (1 - my_x, my_y)

        barrier = pltpu.get_barrier_semaphore()
        for peer in (y_peer, x_peer):
            pl.semaphore_signal(
                barrier, inc=1, device_id=peer,
                device_id_type=pl.DeviceIdType.MESH,
            )
        pl.semaphore_wait(barrier, 2)

        def exchange(slot, partial_f32, send_buf, recv_buf, send_sems,
                     recv_sems, peer):
            send_buf[slot] = partial_f32.astype(bf16)
            rdma = pltpu.make_async_remote_copy(
                src_ref=send_buf.at[slot],
                dst_ref=recv_buf.at[slot],
                send_sem=send_sems.at[slot],
                recv_sem=recv_sems.at[slot],
                device_id=peer,
                device_id_type=pl.DeviceIdType.MESH,
            )
            rdma.start()
            rdma.wait()
            return partial_f32 + recv_buf[slot].astype(jnp.float32)

        acts = x_ref[...].astype(bf16)
        layers = ((win0_ref, wout0_ref), (win1_ref, wout1_ref),
                  (win2_ref, wout2_ref))
        for layer, (win_ref, wout_ref) in enumerate(layers):
            win = win_ref[...].astype(bf16)
            hp = jnp.dot(acts, win, preferred_element_type=jnp.float32)
            h = exchange(layer, hp, h_send, h_recv, h_send_sems,
                         h_recv_sems, y_peer)
            h = jnp.maximum(h, 0.0).astype(bf16)
            wout = wout_ref[...].astype(bf16)
            xp = jnp.dot(h, wout, preferred_element_type=jnp.float32)
            xs = exchange(layer, xp, x_send, x_recv, x_send_sems,
                          x_recv_sems, x_peer)
            acts = xs.astype(bf16)
        out_ref[...] = xs

    return pl.pallas_call(
        body,
        out_shape=jax.ShapeDtypeStruct((b, d_half), jnp.float32),
        in_specs=[pl.BlockSpec(memory_space=pltpu.VMEM)] * 7,
        out_specs=pl.BlockSpec(memory_space=pltpu.VMEM),
        scratch_shapes=[
            pltpu.VMEM((3, b, h_half), bf16),
            pltpu.VMEM((3, b, h_half), bf16),
            pltpu.VMEM((3, b, d_half), bf16),
            pltpu.VMEM((3, b, d_half), bf16),
            pltpu.SemaphoreType.DMA((3,)),
            pltpu.SemaphoreType.DMA((3,)),
            pltpu.SemaphoreType.DMA((3,)),
            pltpu.SemaphoreType.DMA((3,)),
        ],
        compiler_params=pltpu.CompilerParams(
            collective_id=0, vmem_limit_bytes=100 * 1024 * 1024,
        ),
    )(x, Win0, Wout0, Win1, Wout1, Win2, Wout2)


# device time: 38002 ns/iter; 1.4125x vs baseline; 1.3206x over previous
import jax
import jax.numpy as jnp
from jax import lax
from jax.experimental import pallas as pl
from jax.experimental.pallas import tpu as pltpu


def kernel(x, Win0, Wout0, Win1, Wout1, Win2, Wout2):
    b, d_half = x.shape
    h_half = Win0.shape[1]
    bf16 = jnp.bfloat16

    def body(x_ref, win0_ref, wout0_ref, win1_ref, wout1_ref, win2_ref,
             wout2_ref, out_ref,
             win_buf, wout_buf, h_send, h_recv, x_send, x_recv,
             w_sems, h_send_sems, h_recv_sems, x_send_sems, x_recv_sems):
        my_x = lax.axis_index("x")
        my_y = lax.axis_index("y")
        y_peer = (my_x, 1 - my_y)
        x_peer = (1 - my_x, my_y)

        win_hbm = (win0_ref, win1_ref, win2_ref)
        wout_hbm = (wout0_ref, wout1_ref, wout2_ref)

        def win_copy(layer):
            return pltpu.make_async_copy(
                win_hbm[layer], win_buf.at[layer % 2], w_sems.at[2 * layer])

        def wout_copy(layer):
            return pltpu.make_async_copy(
                wout_hbm[layer], wout_buf.at[layer % 2],
                w_sems.at[2 * layer + 1])

        for layer in (0, 1):
            win_copy(layer).start()
            wout_copy(layer).start()

        barrier = pltpu.get_barrier_semaphore()
        for peer in (y_peer, x_peer):
            pl.semaphore_signal(
                barrier, inc=1, device_id=peer,
                device_id_type=pl.DeviceIdType.MESH,
            )
        pl.semaphore_wait(barrier, 2)

        def start_exchange(slot, partial_f32, send_buf, recv_buf, send_sems,
                           recv_sems, peer):
            send_buf[slot] = partial_f32.astype(bf16)
            rdma = pltpu.make_async_remote_copy(
                src_ref=send_buf.at[slot],
                dst_ref=recv_buf.at[slot],
                send_sem=send_sems.at[slot],
                recv_sem=recv_sems.at[slot],
                device_id=peer,
                device_id_type=pl.DeviceIdType.MESH,
            )
            rdma.start()
            return rdma

        acts = x_ref[...].astype(bf16)
        for layer in range(3):
            win_copy(layer).wait()
            win = win_buf[layer % 2].astype(bf16)
            hp = jnp.dot(acts, win, preferred_element_type=jnp.float32)
            if layer == 0:
                win_copy(2).start()
            rdma = start_exchange(layer, hp, h_send, h_recv, h_send_sems,
                                  h_recv_sems, y_peer)
            wout_copy(layer).wait()
            wout = wout_buf[layer % 2].astype(bf16)
            rdma.wait()
            h = hp + h_recv[layer].astype(jnp.float32)
            h = jnp.maximum(h, 0.0).astype(bf16)
            xp = jnp.dot(h, wout, preferred_element_type=jnp.float32)
            if layer == 0:
                wout_copy(2).start()
            rdma = start_exchange(layer, xp, x_send, x_recv, x_send_sems,
                                  x_recv_sems, x_peer)
            rdma.wait()
            xs = xp + x_recv[layer].astype(jnp.float32)
            acts = xs.astype(bf16)
        out_ref[...] = xs

    vmem = pl.BlockSpec(memory_space=pltpu.VMEM)
    hbm = pl.BlockSpec(memory_space=pl.ANY)
    return pl.pallas_call(
        body,
        out_shape=jax.ShapeDtypeStruct((b, d_half), jnp.float32),
        in_specs=[vmem] + [hbm] * 6,
        out_specs=vmem,
        scratch_shapes=[
            pltpu.VMEM((2, d_half, h_half), jnp.float32),
            pltpu.VMEM((2, h_half, d_half), jnp.float32),
            pltpu.VMEM((3, b, h_half), bf16),
            pltpu.VMEM((3, b, h_half), bf16),
            pltpu.VMEM((3, b, d_half), bf16),
            pltpu.VMEM((3, b, d_half), bf16),
            pltpu.SemaphoreType.DMA((6,)),
            pltpu.SemaphoreType.DMA((3,)),
            pltpu.SemaphoreType.DMA((3,)),
            pltpu.SemaphoreType.DMA((3,)),
            pltpu.SemaphoreType.DMA((3,)),
        ],
        compiler_params=pltpu.CompilerParams(
            collective_id=0, vmem_limit_bytes=100 * 1024 * 1024,
        ),
    )(x, Win0, Wout0, Win1, Wout1, Win2, Wout2)


# device time: 37906 ns/iter; 1.4161x vs baseline; 1.0025x over previous
import jax
import jax.numpy as jnp
from jax import lax
from jax.experimental import pallas as pl
from jax.experimental.pallas import tpu as pltpu


def kernel(x, Win0, Wout0, Win1, Wout1, Win2, Wout2):
    b, d_half = x.shape
    h_half = Win0.shape[1]
    bf16 = jnp.bfloat16

    def body(x_ref, win0_ref, wout0_ref, win1_ref, wout1_ref, win2_ref,
             wout2_ref, out_ref,
             win_buf, wout_buf, h_send, h_recv, x_send, x_recv,
             w_sems, h_send_sems, h_recv_sems, x_send_sems, x_recv_sems):
        my_x = lax.axis_index("x")
        my_y = lax.axis_index("y")
        y_peer = (my_x, 1 - my_y)
        x_peer = (1 - my_x, my_y)

        win_hbm = (win0_ref, win1_ref, win2_ref)
        wout_hbm = (wout0_ref, wout1_ref, wout2_ref)

        def win_copy(layer):
            return pltpu.make_async_copy(
                win_hbm[layer], win_buf.at[layer % 2], w_sems.at[2 * layer])

        def wout_copy(layer):
            return pltpu.make_async_copy(
                wout_hbm[layer], wout_buf.at[layer % 2],
                w_sems.at[2 * layer + 1])

        for layer in (0, 1):
            win_copy(layer).start()
            wout_copy(layer).start()

        barrier = pltpu.get_barrier_semaphore()
        for peer in (y_peer, x_peer):
            pl.semaphore_signal(
                barrier, inc=1, device_id=peer,
                device_id_type=pl.DeviceIdType.MESH,
            )
        pl.semaphore_wait(barrier, 2)

        def start_exchange(slot, partial_f32, send_buf, recv_buf, send_sems,
                           recv_sems, peer):
            send_buf[slot] = partial_f32.astype(bf16)
            rdma = pltpu.make_async_remote_copy(
                src_ref=send_buf.at[slot],
                dst_ref=recv_buf.at[slot],
                send_sem=send_sems.at[slot],
                recv_sem=recv_sems.at[slot],
                device_id=peer,
                device_id_type=pl.DeviceIdType.MESH,
            )
            rdma.start()
            return rdma

        acts = x_ref[...].astype(bf16)
        win_copy(0).wait()
        win = win_buf[0].astype(bf16)
        for layer in range(3):
            hp = jnp.dot(acts, win, preferred_element_type=jnp.float32)
            if layer == 0:
                win_copy(2).start()
            rdma = start_exchange(layer, hp, h_send, h_recv, h_send_sems,
                                  h_recv_sems, y_peer)
            wout_copy(layer).wait()
            wout = wout_buf[layer % 2].astype(bf16)
            rdma.wait()
            h = hp + h_recv[layer].astype(jnp.float32)
            h = jnp.maximum(h, 0.0).astype(bf16)
            xp = jnp.dot(h, wout, preferred_element_type=jnp.float32)
            if layer == 0:
                wout_copy(2).start()
            rdma = start_exchange(layer, xp, x_send, x_recv, x_send_sems,
                                  x_recv_sems, x_peer)
            if layer < 2:
                win_copy(layer + 1).wait()
                win = win_buf[(layer + 1) % 2].astype(bf16)
            rdma.wait()
            xs = xp + x_recv[layer].astype(jnp.float32)
            acts = xs.astype(bf16)
        out_ref[...] = xs

    vmem = pl.BlockSpec(memory_space=pltpu.VMEM)
    hbm = pl.BlockSpec(memory_space=pl.ANY)
    return pl.pallas_call(
        body,
        out_shape=jax.ShapeDtypeStruct((b, d_half), jnp.float32),
        in_specs=[vmem] + [hbm] * 6,
        out_specs=vmem,
        scratch_shapes=[
            pltpu.VMEM((2, d_half, h_half), jnp.float32),
            pltpu.VMEM((2, h_half, d_half), jnp.float32),
            pltpu.VMEM((3, b, h_half), bf16),
            pltpu.VMEM((3, b, h_half), bf16),
            pltpu.VMEM((3, b, d_half), bf16),
            pltpu.VMEM((3, b, d_half), bf16),
            pltpu.SemaphoreType.DMA((6,)),
            pltpu.SemaphoreType.DMA((3,)),
            pltpu.SemaphoreType.DMA((3,)),
            pltpu.SemaphoreType.DMA((3,)),
            pltpu.SemaphoreType.DMA((3,)),
        ],
        compiler_params=pltpu.CompilerParams(
            collective_id=0, vmem_limit_bytes=100 * 1024 * 1024,
        ),
    )(x, Win0, Wout0, Win1, Wout1, Win2, Wout2)


# device time: 37890 ns/iter; 1.4167x vs baseline; 1.0004x over previous
import jax
import jax.numpy as jnp
from jax import lax
from jax.experimental import pallas as pl
from jax.experimental.pallas import tpu as pltpu


def kernel(x, Win0, Wout0, Win1, Wout1, Win2, Wout2):
    b, d_half = x.shape
    h_half = Win0.shape[1]
    bf16 = jnp.bfloat16

    def body(x_ref, win0_ref, wout0_ref, win1_ref, wout1_ref, win2_ref,
             wout2_ref, out_ref,
             win_buf, wout_buf, h_send, h_recv, x_send, x_recv,
             w_sems, h_send_sems, h_recv_sems, x_send_sems, x_recv_sems):
        my_x = lax.axis_index("x")
        my_y = lax.axis_index("y")
        y_peer = (my_x, 1 - my_y)
        x_peer = (1 - my_x, my_y)

        win_hbm = (win0_ref, win1_ref, win2_ref)
        wout_hbm = (wout0_ref, wout1_ref, wout2_ref)

        def win_copy(layer):
            return pltpu.make_async_copy(
                win_hbm[layer], win_buf.at[layer % 2], w_sems.at[2 * layer])

        def wout_copy(layer):
            return pltpu.make_async_copy(
                wout_hbm[layer], wout_buf.at[layer % 2],
                w_sems.at[2 * layer + 1])

        for layer in (0, 1):
            win_copy(layer).start()
            wout_copy(layer).start()

        barrier = pltpu.get_barrier_semaphore()
        for peer in (y_peer, x_peer):
            pl.semaphore_signal(
                barrier, inc=1, device_id=peer,
                device_id_type=pl.DeviceIdType.MESH,
            )
        pl.semaphore_wait(barrier, 2)

        def start_exchange(slot, partial_f32, send_buf, recv_buf, send_sems,
                           recv_sems, peer):
            send_buf[slot] = partial_f32.astype(bf16)
            rdma = pltpu.make_async_remote_copy(
                src_ref=send_buf.at[slot],
                dst_ref=recv_buf.at[slot],
                send_sem=send_sems.at[slot],
                recv_sem=recv_sems.at[slot],
                device_id=peer,
                device_id_type=pl.DeviceIdType.MESH,
            )
            rdma.start()
            return rdma

        acts = x_ref[...]
        win_copy(0).wait()
        for layer in range(3):
            hp = jnp.dot(acts, win_buf[layer % 2],
                         preferred_element_type=jnp.float32)
            if layer == 0:
                win_copy(2).start()
            rdma = start_exchange(layer, hp, h_send, h_recv, h_send_sems,
                                  h_recv_sems, y_peer)
            wout_copy(layer).wait()
            rdma.wait()
            h = hp + h_recv[layer].astype(jnp.float32)
            h = jnp.maximum(h, 0.0)
            xp = jnp.dot(h, wout_buf[layer % 2],
                         preferred_element_type=jnp.float32)
            if layer == 0:
                wout_copy(2).start()
            rdma = start_exchange(layer, xp, x_send, x_recv, x_send_sems,
                                  x_recv_sems, x_peer)
            if layer < 2:
                win_copy(layer + 1).wait()
            rdma.wait()
            acts = xp + x_recv[layer].astype(jnp.float32)
        out_ref[...] = acts

    vmem = pl.BlockSpec(memory_space=pltpu.VMEM)
    hbm = pl.BlockSpec(memory_space=pl.ANY)
    return pl.pallas_call(
        body,
        out_shape=jax.ShapeDtypeStruct((b, d_half), jnp.float32),
        in_specs=[vmem] + [hbm] * 6,
        out_specs=vmem,
        scratch_shapes=[
            pltpu.VMEM((2, d_half, h_half), jnp.float32),
            pltpu.VMEM((2, h_half, d_half), jnp.float32),
            pltpu.VMEM((3, b, h_half), bf16),
            pltpu.VMEM((3, b, h_half), bf16),
            pltpu.VMEM((3, b, d_half), bf16),
            pltpu.VMEM((3, b, d_half), bf16),
            pltpu.SemaphoreType.DMA((6,)),
            pltpu.SemaphoreType.DMA((3,)),
            pltpu.SemaphoreType.DMA((3,)),
            pltpu.SemaphoreType.DMA((3,)),
            pltpu.SemaphoreType.DMA((3,)),
        ],
        compiler_params=pltpu.CompilerParams(
            collective_id=0, vmem_limit_bytes=100 * 1024 * 1024,
        ),
    )(x, Win0, Wout0, Win1, Wout1, Win2, Wout2)


# device time: 37538 ns/iter; 1.4300x vs baseline; 1.0094x over previous
import jax
import jax.numpy as jnp
from jax import lax
from jax.experimental import pallas as pl
from jax.experimental.pallas import tpu as pltpu

N_PARTS = 4


def kernel(x, Win0, Wout0, Win1, Wout1, Win2, Wout2):
    b, d_half = x.shape
    h_half = Win0.shape[1]
    bf16 = jnp.bfloat16

    def body(x_ref, win0_ref, wout0_ref, win1_ref, wout1_ref, win2_ref,
             wout2_ref, out_ref,
             win_buf, wout_buf, h_send, h_recv, x_send, x_recv,
             w_sems, h_send_sems, h_recv_sems, x_send_sems, x_recv_sems):
        my_x = lax.axis_index("x")
        my_y = lax.axis_index("y")
        y_peer = (my_x, 1 - my_y)
        x_peer = (1 - my_x, my_y)

        win_hbm = (win0_ref, win1_ref, win2_ref)
        wout_hbm = (wout0_ref, wout1_ref, wout2_ref)

        def w_parts(seq, hbm_ref, buf, slot, nrows):
            q = nrows // N_PARTS
            bank = (seq % 2) * N_PARTS
            return [
                pltpu.make_async_copy(
                    hbm_ref.at[pl.ds(c * q, q)],
                    buf.at[slot, pl.ds(c * q, q)],
                    w_sems.at[bank + c],
                )
                for c in range(N_PARTS)
            ]

        def win_parts(layer):
            return w_parts(2 * layer, win_hbm[layer], win_buf,
                           layer % 2, d_half)

        def wout_parts(layer):
            return w_parts(2 * layer + 1, wout_hbm[layer], wout_buf,
                           layer % 2, h_half)

        def start_all(parts):
            for p in parts:
                p.start()

        def wait_all(parts):
            for p in parts:
                p.wait()

        start_all(win_parts(0))

        barrier = pltpu.get_barrier_semaphore()
        for peer in (y_peer, x_peer):
            pl.semaphore_signal(
                barrier, inc=1, device_id=peer,
                device_id_type=pl.DeviceIdType.MESH,
            )
        pl.semaphore_wait(barrier, 2)

        def start_exchange(slot, partial_f32, send_buf, recv_buf, send_sems,
                           recv_sems, peer):
            send_buf[slot] = partial_f32.astype(bf16)
            rdma = pltpu.make_async_remote_copy(
                src_ref=send_buf.at[slot],
                dst_ref=recv_buf.at[slot],
                send_sem=send_sems.at[slot],
                recv_sem=recv_sems.at[slot],
                device_id=peer,
                device_id_type=pl.DeviceIdType.MESH,
            )
            rdma.start()
            return rdma

        acts = x_ref[...]
        wait_all(win_parts(0))
        start_all(wout_parts(0))
        for layer in range(3):
            hp = jnp.dot(acts, win_buf[layer % 2],
                         preferred_element_type=jnp.float32)
            rdma = start_exchange(layer, hp, h_send, h_recv, h_send_sems,
                                  h_recv_sems, y_peer)
            wait_all(wout_parts(layer))
            if layer < 2:
                start_all(win_parts(layer + 1))
            rdma.wait()
            h = hp + h_recv[layer].astype(jnp.float32)
            h = jnp.maximum(h, 0.0)
            xp = jnp.dot(h, wout_buf[layer % 2],
                         preferred_element_type=jnp.float32)
            rdma = start_exchange(layer, xp, x_send, x_recv, x_send_sems,
                                  x_recv_sems, x_peer)
            if layer < 2:
                wait_all(win_parts(layer + 1))
                start_all(wout_parts(layer + 1))
            rdma.wait()
            acts = xp + x_recv[layer].astype(jnp.float32)
        out_ref[...] = acts

    vmem = pl.BlockSpec(memory_space=pltpu.VMEM)
    hbm = pl.BlockSpec(memory_space=pl.ANY)
    return pl.pallas_call(
        body,
        out_shape=jax.ShapeDtypeStruct((b, d_half), jnp.float32),
        in_specs=[vmem] + [hbm] * 6,
        out_specs=vmem,
        scratch_shapes=[
            pltpu.VMEM((2, d_half, h_half), jnp.float32),
            pltpu.VMEM((2, h_half, d_half), jnp.float32),
            pltpu.VMEM((3, b, h_half), bf16),
            pltpu.VMEM((3, b, h_half), bf16),
            pltpu.VMEM((3, b, d_half), bf16),
            pltpu.VMEM((3, b, d_half), bf16),
            pltpu.SemaphoreType.DMA((2 * N_PARTS,)),
            pltpu.SemaphoreType.DMA((3,)),
            pltpu.SemaphoreType.DMA((3,)),
            pltpu.SemaphoreType.DMA((3,)),
            pltpu.SemaphoreType.DMA((3,)),
        ],
        compiler_params=pltpu.CompilerParams(
            collective_id=0, vmem_limit_bytes=100 * 1024 * 1024,
        ),
    )(x, Win0, Wout0, Win1, Wout1, Win2, Wout2)
